# baseline (speedup 1.0000x reference)
import functools

import jax
import jax.numpy as jnp
from jax import lax
from jax.experimental import pallas as pl
from jax.experimental.pallas import tpu as pltpu
from jax.experimental.pallas import tpu_sc as plsc

NBUF = 4
LA = 3


@functools.cache
def _make_gather(v, d, bsz, hist):
    info = plsc.get_sparse_core_info()
    nc, ns = info.num_cores, info.num_subcores
    nw = nc * ns
    assert bsz % (128 * nw) == 0 and hist % 8 == 0 and d % 8 == 0
    nbc = bsz // 128
    nhr = hist // 8
    nfb = d // 8
    passes = nbc // nw
    ngrp = hist // 2
    assert ngrp % NBUF == 0 and ngrp >= 2 * NBUF
    mesh = plsc.VectorSubcoreMesh(core_axis_name="c", subcore_axis_name="s")

    @functools.partial(
        pl.kernel,
        mesh=mesh,
        out_type=jax.ShapeDtypeStruct((hist, nfb, nbc, 8, 128), jnp.float32),
        compiler_params=pltpu.CompilerParams(
            use_tc_tiling_on_sc=False, needs_layout_passes=False),
        scratch_types=(
            [pltpu.VMEM((nhr, 8, 128), jnp.int32)]
            + [pltpu.VMEM((256, d), jnp.float32) for _ in range(NBUF)]
            + [pltpu.VMEM((2, nfb, 8, 128), jnp.float32) for _ in range(NBUF)]
            + [pltpu.SemaphoreType.DMA for _ in range(2 * NBUF)]
        ),
    )
    def gather(table_hbm, x4_hbm, out_hbm, idx_v, *bufs):
        rbufs = bufs[:NBUF]
        tbufs = bufs[NBUF:2 * NBUF]
        gsems = bufs[2 * NBUF:3 * NBUF]
        ssems = bufs[3 * NBUF:]
        wid = lax.axis_index("s") * nc + lax.axis_index("c")
        lanes = lax.iota(jnp.int32, 16)
        rowv = [16 * k + lanes for k in range(8)]

        def g_copy(h, b):
            return pltpu.make_async_copy(
                table_hbm.at[idx_v.at[h // 8, h % 8]],
                rbufs[b].at[pl.ds(128 * (h % 2), 128)], gsems[b])

        def g_group(j, b):
            g_copy(2 * j, b).start()
            g_copy(2 * j + 1, b).start()

        def s_copy(j, b, bc):
            return pltpu.make_async_copy(
                tbufs[b], out_hbm.at[pl.ds(2 * j, 2), :, bc], ssems[b])

        def transpose(b):
            rbuf, tbuf = rbufs[b], tbufs[b]
            for i in range(2):
                for fb in range(nfb):
                    for fi in range(8):
                        col = jnp.full((16,), fb * 8 + fi, jnp.int32)
                        vecs = [plsc.load_gather(rbuf, [128 * i + rowv[k], col])
                                for k in range(8)]
                        for k in range(8):
                            tbuf[i, fb, fi, pl.ds(16 * k, 16)] = vecs[k]

        def process(j, b, bc):
            g_copy(2 * j, b).wait()
            g_copy(2 * j + 1, b).wait()

            @pl.when(j + LA < ngrp)
            def _():
                g_group(j + LA, (b + LA) % NBUF)

            @pl.when(j >= NBUF)
            def _():
                s_copy(j - NBUF, b, bc).wait()

            transpose(b)
            s_copy(j, b, bc).start()

        def one_pass(bc):
            pltpu.sync_copy(x4_hbm.at[:, bc], idx_v)
            for j in range(LA):
                g_group(j, j % NBUF)

            def body(g, carry):
                for p in range(NBUF):
                    process(NBUF * g + p, p, bc)
                return carry

            lax.fori_loop(0, ngrp // NBUF, body, 0, unroll=False)
            for j in range(ngrp - NBUF, ngrp):
                s_copy(j, j % NBUF, bc).wait()

        for i in range(passes):
            one_pass(wid * passes + i)

    return gather


def kernel(x, emb):
    bsz, hist = x.shape
    v, d = emb.shape
    x4 = x.T.reshape(hist // 8, 8, bsz // 128, 128).transpose(0, 2, 1, 3)
    x4 = x4.astype(jnp.int32)
    out4 = _make_gather(v, d, bsz, hist)(emb, x4)
    return out4.transpose(2, 4, 0, 1, 3).reshape(bsz, hist, d)


# speedup vs baseline: 1.1498x; 1.1498x over previous
import functools

import jax
import jax.numpy as jnp
from jax import lax
from jax.experimental import pallas as pl
from jax.experimental.pallas import tpu as pltpu
from jax.experimental.pallas import tpu_sc as plsc

CHUNK = 640
NBUF = 4
LOOKAHEAD = 2


@functools.cache
def _make_gather(n, d):
    info = plsc.get_sparse_core_info()
    nc, ns = info.num_cores, info.num_subcores
    nw = nc * ns
    b_per_w = n // nw
    n_chunks = b_per_w // CHUNK
    groups = n_chunks // NBUF
    assert b_per_w * nw == n and n_chunks * CHUNK == b_per_w
    assert groups * NBUF == n_chunks and groups >= 3
    mesh = plsc.VectorSubcoreMesh(core_axis_name="c", subcore_axis_name="s")

    @functools.partial(
        pl.kernel,
        mesh=mesh,
        out_type=jax.ShapeDtypeStruct((nw * b_per_w, d), jnp.float32),
        compiler_params=pltpu.CompilerParams(use_tc_tiling_on_sc=False),
        scratch_types=(
            [pltpu.VMEM((n_chunks, CHUNK), jnp.int32)]
            + [pltpu.VMEM((CHUNK, d), jnp.float32) for _ in range(NBUF)]
            + [pltpu.SemaphoreType.DMA for _ in range(2 * NBUF)]
        ),
    )
    def gather(table_hbm, idx_hbm, out_hbm, idx_v, *scratch):
        rbufs = scratch[:NBUF]
        gsems = scratch[NBUF:2 * NBUF]
        ssems = scratch[2 * NBUF:]
        wid = lax.axis_index("s") * nc + lax.axis_index("c")
        base = wid * b_per_w

        pltpu.sync_copy(idx_hbm.at[pl.ds(wid * n_chunks, n_chunks)], idx_v)

        def g_copy(c, b):
            return pltpu.make_async_copy(
                table_hbm.at[idx_v.at[c]], rbufs[b], gsems[b])

        def s_copy(c, b):
            return pltpu.make_async_copy(
                rbufs[b], out_hbm.at[pl.ds(base + c * CHUNK, CHUNK)], ssems[b])

        for c in range(LOOKAHEAD):
            g_copy(c, c % NBUF).start()

        for b in range(NBUF):
            c = b
            if c + LOOKAHEAD >= NBUF:
                s_copy(c - LOOKAHEAD, (c + LOOKAHEAD) % NBUF).wait()
            g_copy(c + LOOKAHEAD, (c + LOOKAHEAD) % NBUF).start()
            g_copy(c, b).wait()
            s_copy(c, b).start()

        def body(g, carry):
            for b in range(NBUF):
                c = g * NBUF + b
                bg = (b + LOOKAHEAD) % NBUF
                s_copy(c - LOOKAHEAD, bg).wait()
                g_copy(c + LOOKAHEAD, bg).start()
                g_copy(c, b).wait()
                s_copy(c, b).start()
            return carry

        lax.fori_loop(1, groups - 1, body, 0)

        for b in range(NBUF):
            c = (groups - 1) * NBUF + b
            if c + LOOKAHEAD < n_chunks:
                bg = (b + LOOKAHEAD) % NBUF
                s_copy(c - LOOKAHEAD, bg).wait()
                g_copy(c + LOOKAHEAD, bg).start()
            g_copy(c, b).wait()
            s_copy(c, b).start()

        for b in range(NBUF):
            c = n_chunks - NBUF + b
            s_copy(c, b).wait()

    return gather


def kernel(x, emb):
    bsz, hist = x.shape
    d = emb.shape[1]
    n = x.size
    idx = x.T.reshape(n // CHUNK, CHUNK).astype(jnp.int32)
    out = _make_gather(n, d)(emb, idx)
    return out.reshape(hist, bsz, d).transpose(1, 0, 2)
